# baseline (speedup 1.0000x reference)
import functools

import jax
import jax.numpy as jnp
from jax import lax
from jax.experimental import pallas as pl
from jax.experimental.pallas import tpu as pltpu
from jax.experimental.pallas import tpu_sc as plsc

NC, NS = 2, 16
NW = NC * NS
C = 32


def _sc_embed(level_ids_flat, token_ids_flat, emb, level_embed):
    n = token_ids_flat.shape[0]
    d = emb.shape[1]
    rpw = n // NW
    nchunk = rpw // C
    mesh = plsc.VectorSubcoreMesh(core_axis_name="c", subcore_axis_name="s",
                                  num_cores=NC, num_subcores=NS)

    @functools.partial(
        pl.kernel,
        out_type=jax.ShapeDtypeStruct((n, d), jnp.float32),
        mesh=mesh,
        scratch_types=[
            pltpu.VMEM((C,), jnp.int32),
            pltpu.VMEM((C,), jnp.int32),
            pltpu.VMEM((C, d), jnp.float32),
            pltpu.VMEM((C, d), jnp.float32),
            pltpu.SemaphoreType.DMA,
            pltpu.SemaphoreType.DMA,
        ],
    )
    def k(lvl_hbm, tok_hbm, emb_hbm, lev_hbm, out_hbm,
          tok_idx, lvl_idx, rows, lrows, sem1, sem2):
        wid = lax.axis_index("s") * NC + lax.axis_index("c")
        wbase = wid * rpw

        def chunk_body(c, carry):
            base = wbase + c * C
            pltpu.sync_copy(tok_hbm.at[pl.ds(base, C)], tok_idx)
            pltpu.sync_copy(lvl_hbm.at[pl.ds(base, C)], lvl_idx)
            cp1 = pltpu.async_copy(emb_hbm.at[tok_idx], rows, sem1)
            cp2 = pltpu.async_copy(lev_hbm.at[lvl_idx], lrows, sem2)
            cp1.wait()
            cp2.wait()

            def row_body(i, rcarry):
                for j in range(d // 16):
                    sl = pl.ds(j * 16, 16)
                    rows[i, sl] = rows[i, sl] + lrows[i, sl]
                return rcarry

            lax.fori_loop(0, C, row_body, 0)
            pltpu.sync_copy(rows, out_hbm.at[pl.ds(base, C)])
            return carry

        lax.fori_loop(0, nchunk, chunk_body, 0)

    return k(level_ids_flat, token_ids_flat, emb, level_embed)


def kernel(level_ids, token_ids, emb, level_embed):
    b, s = token_ids.shape
    n = b * s
    out = _sc_embed(level_ids.reshape(n), token_ids.reshape(n),
                    emb, level_embed)
    return out.reshape(b, s, emb.shape[1])


# speedup vs baseline: 2.3466x; 2.3466x over previous
import functools

import jax
import jax.numpy as jnp
from jax import lax
from jax.experimental import pallas as pl
from jax.experimental.pallas import tpu as pltpu
from jax.experimental.pallas import tpu_sc as plsc

NC, NS = 2, 16
NW = NC * NS
C = 16
NB = 4


def _sc_embed(level_ids_flat, token_ids_flat, emb, level_embed):
    n = token_ids_flat.shape[0]
    d = emb.shape[1]
    nlev = level_embed.shape[0]
    rpw = n // NW
    nchunk = rpw // C
    nround = nchunk // NB
    mesh = plsc.VectorSubcoreMesh(core_axis_name="c", subcore_axis_name="s",
                                  num_cores=NC, num_subcores=NS)

    @functools.partial(
        pl.kernel,
        out_type=jax.ShapeDtypeStruct((n, d), jnp.float32),
        mesh=mesh,
        scratch_types=[
            pltpu.VMEM((NB, C), jnp.int32),
            pltpu.VMEM((NB, C), jnp.int32),
            pltpu.VMEM((NB, C, d), jnp.float32),
            pltpu.VMEM((nlev, d), jnp.float32),
            pltpu.SemaphoreType.DMA((NB,)),
            pltpu.SemaphoreType.DMA((NB,)),
        ],
    )
    def k(lvl_hbm, tok_hbm, emb_hbm, lev_hbm, out_hbm,
          tok_idx, lvl_idx, rows, lev_v, sem_g, sem_s):
        wid = lax.axis_index("s") * NC + lax.axis_index("c")
        wbase = wid * rpw

        pltpu.sync_copy(lev_hbm, lev_v)

        def issue_gather(b, c):
            base = wbase + c * C
            pltpu.sync_copy(tok_hbm.at[pl.ds(base, C)], tok_idx.at[b])
            pltpu.sync_copy(lvl_hbm.at[pl.ds(base, C)], lvl_idx.at[b])
            pltpu.async_copy(emb_hbm.at[tok_idx.at[b]], rows.at[b],
                             sem_g.at[b])

        def wait_gather(b):
            pltpu.make_async_copy(emb_hbm.at[tok_idx.at[b]], rows.at[b],
                                  sem_g.at[b]).wait()

        def issue_store(b, c):
            base = wbase + c * C
            pltpu.async_copy(rows.at[b], out_hbm.at[pl.ds(base, C)],
                             sem_s.at[b])

        def wait_store(b):
            pltpu.make_async_copy(rows.at[b], out_hbm.at[pl.ds(0, C)],
                                  sem_s.at[b]).wait()

        issue_gather(0, 0)
        issue_gather(1, 1)

        def round_body(g, carry):
            for b in range(NB):
                c = g * NB + b
                wait_gather(b)

                v = lvl_idx[b, pl.ds(0, 16)]
                levs = [v[t] for t in range(16)]

                def j_body(j, jcarry, _b=b, _levs=levs):
                    sl = pl.ds(j * 16, 16)
                    for i in range(C):
                        plsc.addupdate(rows.at[_b, i, sl],
                                       lev_v[_levs[i], sl])
                    return jcarry

                lax.fori_loop(0, d // 16, j_body, 0)
                issue_store(b, c)

                bn = (b + 2) % NB

                @pl.when(c + 2 >= NB)
                def _():
                    wait_store(bn)

                @pl.when(c + 2 < nchunk)
                def _():
                    issue_gather(bn, c + 2)
            return carry

        lax.fori_loop(0, nround, round_body, 0)
        wait_store((nchunk - 2) % NB)
        wait_store((nchunk - 1) % NB)

    return k(level_ids_flat, token_ids_flat, emb, level_embed)


def kernel(level_ids, token_ids, emb, level_embed):
    b, s = token_ids.shape
    n = b * s
    out = _sc_embed(level_ids.reshape(n), token_ids.reshape(n),
                    emb, level_embed)
    return out.reshape(b, s, emb.shape[1])


# speedup vs baseline: 2.7838x; 1.1864x over previous
import functools

import jax
import jax.numpy as jnp
from jax import lax
from jax.experimental import pallas as pl
from jax.experimental.pallas import tpu as pltpu
from jax.experimental.pallas import tpu_sc as plsc

NC, NS = 2, 16
NW = NC * NS
C = 16
NB = 4
LOOK = NB - 2


def _sc_embed(level_ids_flat, token_ids_flat, emb, level_embed):
    n = token_ids_flat.shape[0]
    d = emb.shape[1]
    nlev = level_embed.shape[0]
    rpw = n // NW
    nchunk = rpw // C
    nround = nchunk // NB
    mesh = plsc.VectorSubcoreMesh(core_axis_name="c", subcore_axis_name="s",
                                  num_cores=NC, num_subcores=NS)

    @functools.partial(
        pl.kernel,
        out_type=jax.ShapeDtypeStruct((n, d), jnp.float32),
        mesh=mesh,
        scratch_types=[
            pltpu.VMEM((rpw,), jnp.int32),
            pltpu.VMEM((rpw,), jnp.int32),
            pltpu.VMEM((NB, C, d), jnp.float32),
            pltpu.VMEM((nlev, d), jnp.float32),
            pltpu.SemaphoreType.DMA((NB,)),
            pltpu.SemaphoreType.DMA((NB,)),
        ],
    )
    def k(lvl_hbm, tok_hbm, emb_hbm, lev_hbm, out_hbm,
          tok_v, lvl_v, rows, lev_v, sem_g, sem_s):
        wid = lax.axis_index("s") * NC + lax.axis_index("c")
        wbase = wid * rpw

        pltpu.sync_copy(tok_hbm.at[pl.ds(wbase, rpw)], tok_v)
        pltpu.sync_copy(lvl_hbm.at[pl.ds(wbase, rpw)], lvl_v)
        pltpu.sync_copy(lev_hbm, lev_v)

        def issue_gather(b, c):
            pltpu.async_copy(emb_hbm.at[tok_v.at[pl.ds(c * C, C)]],
                             rows.at[b], sem_g.at[b])

        def wait_gather(b):
            pltpu.make_async_copy(emb_hbm.at[tok_v.at[pl.ds(0, C)]],
                                  rows.at[b], sem_g.at[b]).wait()

        def issue_store(b, c):
            base = wbase + c * C
            pltpu.async_copy(rows.at[b], out_hbm.at[pl.ds(base, C)],
                             sem_s.at[b])

        def wait_store(b):
            pltpu.make_async_copy(rows.at[b], out_hbm.at[pl.ds(0, C)],
                                  sem_s.at[b]).wait()

        for c0 in range(LOOK):
            issue_gather(c0, c0)

        def round_body(g, carry):
            for b in range(NB):
                c = g * NB + b
                wait_gather(b)

                v = lvl_v[pl.ds(c * C, 16)]
                levs = [v[t] for t in range(16)]

                def j_body(j, jcarry, _b=b, _levs=levs):
                    sl = pl.ds(j * 16, 16)
                    for i in range(C):
                        plsc.addupdate(rows.at[_b, i, sl],
                                       lev_v[_levs[i], sl])
                    return jcarry

                lax.fori_loop(0, d // 16, j_body, 0)
                issue_store(b, c)

                bn = (b + LOOK) % NB

                @pl.when(c + LOOK >= NB)
                def _():
                    wait_store(bn)

                @pl.when(c + LOOK < nchunk)
                def _():
                    issue_gather(bn, c + LOOK)
            return carry

        lax.fori_loop(0, nround, round_body, 0)
        for ct in range(nchunk - NB + LOOK, nchunk):
            wait_store(ct % NB)

    return k(level_ids_flat, token_ids_flat, emb, level_embed)


def kernel(level_ids, token_ids, emb, level_embed):
    b, s = token_ids.shape
    n = b * s
    out = _sc_embed(level_ids.reshape(n), token_ids.reshape(n),
                    emb, level_embed)
    return out.reshape(b, s, emb.shape[1])


# speedup vs baseline: 4.9045x; 1.7618x over previous
import functools

import jax
import jax.numpy as jnp
from jax import lax
from jax.experimental import pallas as pl
from jax.experimental.pallas import tpu as pltpu
from jax.experimental.pallas import tpu_sc as plsc

NC, NS = 2, 16
NW = NC * NS
C = 16
NB = 4
LOOK = NB - 2


def _sc_embed(level_ids_flat, token_ids_flat, emb, level_embed):
    n = token_ids_flat.shape[0]
    d = emb.shape[1]
    nlev = level_embed.shape[0]
    rpw = n // NW
    nchunk = rpw // C
    nround = nchunk // NB
    mesh = plsc.VectorSubcoreMesh(core_axis_name="c", subcore_axis_name="s",
                                  num_cores=NC, num_subcores=NS)

    @functools.partial(
        pl.kernel,
        out_type=jax.ShapeDtypeStruct((n, d), jnp.float32),
        mesh=mesh,
        scratch_types=[
            pltpu.VMEM((rpw,), jnp.int32),
            pltpu.VMEM((rpw,), jnp.int32),
            pltpu.VMEM((NB, C, d), jnp.float32),
            pltpu.VMEM((nlev, d), jnp.float32),
            pltpu.VMEM_SHARED((nlev, d), jnp.float32),
            pltpu.SemaphoreType.DMA((NB,)),
            pltpu.SemaphoreType.DMA((NB,)),
        ],
    )
    def k(lvl_hbm, tok_hbm, emb_hbm, lev_hbm, out_hbm,
          tok_v, lvl_v, rows, lev_v, lev_sh, sem_g, sem_s):
        wid = lax.axis_index("s") * NC + lax.axis_index("c")
        wbase = wid * rpw

        pltpu.sync_copy(tok_hbm.at[pl.ds(wbase, rpw)], tok_v)
        pltpu.sync_copy(lvl_hbm.at[pl.ds(wbase, rpw)], lvl_v)
        pltpu.sync_copy(lev_hbm, lev_v)

        @pl.when(lax.axis_index("s") == 0)
        def _():
            pltpu.sync_copy(lev_v, lev_sh)

        plsc.subcore_barrier()

        def issue_gather(b, c):
            pltpu.async_copy(emb_hbm.at[tok_v.at[pl.ds(c * C, C)]],
                             rows.at[b], sem_g.at[b])

        def wait_gather(b):
            pltpu.make_async_copy(emb_hbm.at[tok_v.at[pl.ds(0, C)]],
                                  rows.at[b], sem_g.at[b]).wait()

        def issue_store(b, c):
            base = wbase + c * C
            pltpu.async_copy(rows.at[b], out_hbm.at[pl.ds(base, C)],
                             sem_s.at[b])

        def wait_store(b):
            pltpu.make_async_copy(rows.at[b], out_hbm.at[pl.ds(0, C)],
                                  sem_s.at[b]).wait()

        for c0 in range(LOOK):
            issue_gather(c0, c0)

        def round_body(g, carry):
            for b in range(NB):
                c = g * NB + b
                wait_gather(b)

                v = lvl_v[pl.ds(c * C, 16)]
                levs = [v[t] for t in range(16)]

                @plsc.parallel_loop(0, d // 16, unroll=2)
                def _(j, _b=b, _levs=levs):
                    sl = pl.ds(j * 16, 16)
                    for i in range(C):
                        plsc.addupdate(rows.at[_b, i, sl],
                                       lev_v[_levs[i], sl])

                issue_store(b, c)

                bn = (b + LOOK) % NB

                @pl.when(c + LOOK >= NB)
                def _():
                    wait_store(bn)

                @pl.when(c + LOOK < nchunk)
                def _():
                    issue_gather(bn, c + LOOK)
            return carry

        lax.fori_loop(0, nround, round_body, 0)
        for ct in range(nchunk - NB + LOOK, nchunk):
            wait_store(ct % NB)

    return k(level_ids_flat, token_ids_flat, emb, level_embed)


def kernel(level_ids, token_ids, emb, level_embed):
    b, s = token_ids.shape
    n = b * s
    out = _sc_embed(level_ids.reshape(n), token_ids.reshape(n),
                    emb, level_embed)
    return out.reshape(b, s, emb.shape[1])
